# Initial kernel scaffold; baseline (speedup 1.0000x reference)
import jax
import jax.numpy as jnp
from jax.experimental import pallas as pl


def kernel(x, edge_index, W, att_src, att_dst, bias):
    raise NotImplementedError("write your pallas kernel here")


# speedup vs baseline: 45.8696x; 45.8696x over previous
import functools

import jax
import jax.numpy as jnp
from jax import lax
from jax.experimental import pallas as pl
from jax.experimental.pallas import tpu as pltpu
from jax.experimental.pallas import tpu_sc as plsc

N = 100000
E = 1600000
D_IN = 34
H = 8
F = 8
HF = H * F

NC, NS = 2, 16
NT = NC * NS
SB = 128
GRP = 8
EG = SB * GRP
G0 = 49
G1 = 2 * G0
E_PAD = NT * G0 * EG
SB_REAL = E // SB
NSB = E_PAD // SB
NPT = 6256
NPAD = NS * NPT

_mesh = plsc.VectorSubcoreMesh(core_axis_name="c", subcore_axis_name="s",
                               num_cores=NC, num_subcores=NS)


def _dense_body(x_ref, w_ref, m_ref, h_ref, ad_ref, wl_ref):
    h = jnp.dot(x_ref[...], w_ref[...], preferred_element_type=jnp.float32)
    h_ref[...] = h
    ad = jnp.dot(h, m_ref[...], preferred_element_type=jnp.float32)
    ad_ref[...] = ad
    e = ad[:, :H] + ad[:, H:]
    wl_ref[...] = jnp.exp(jnp.maximum(e, 0.2 * e))


def _dense_prologue(x, W, Mcat):
    blk = 1000
    return pl.pallas_call(
        _dense_body,
        grid=(N // blk,),
        in_specs=[
            pl.BlockSpec((blk, D_IN), lambda i: (i, 0)),
            pl.BlockSpec((D_IN, HF), lambda i: (0, 0)),
            pl.BlockSpec((HF, 2 * H), lambda i: (0, 0)),
        ],
        out_specs=[
            pl.BlockSpec((blk, HF), lambda i: (i, 0)),
            pl.BlockSpec((blk, 2 * H), lambda i: (i, 0)),
            pl.BlockSpec((blk, H), lambda i: (i, 0)),
        ],
        out_shape=[
            jax.ShapeDtypeStruct((N, HF), jnp.float32),
            jax.ShapeDtypeStruct((N, 2 * H), jnp.float32),
            jax.ShapeDtypeStruct((N, H), jnp.float32),
        ],
    )(x, W, Mcat)


@functools.partial(
    pl.kernel,
    out_type=[
        jax.ShapeDtypeStruct((H, NSB, SB), jnp.float32),
        jax.ShapeDtypeStruct((NC, NPAD, 16), jnp.float32),
    ],
    mesh=_mesh,
    compiler_params=pltpu.CompilerParams(needs_layout_passes=False, use_tc_tiling_on_sc=False),
    scratch_types=[
        pltpu.VMEM_SHARED((NPAD, 16), jnp.float32),
        pltpu.VMEM((GRP, SB), jnp.int32),
        pltpu.VMEM((GRP, SB), jnp.int32),
        pltpu.VMEM((SB, 16), jnp.float32),
        pltpu.VMEM((SB, 16), jnp.float32),
        pltpu.VMEM((SB, 16), jnp.float32),
        pltpu.VMEM((H, GRP, SB), jnp.float32),
        pltpu.SemaphoreType.DMA,
    ],
)
def _sc_pass0(ad_hbm, src_hbm, dst_hbm, zero_hbm, wt_hbm, s_hbm,
              acc_sh, srcbuf, dstbuf, arows, brows, wbuf, wtbuf, sem):
    c = lax.axis_index("c")
    s = lax.axis_index("s")
    tid = c * NS + s
    pltpu.sync_copy(zero_hbm, acc_sh.at[pl.ds(s * NPT, NPT)])
    pltpu.sync_copy(zero_hbm.at[pl.ds(0, SB)], wbuf)
    plsc.subcore_barrier()

    lane = lax.iota(jnp.int32, 16)

    def group_body(g, carry):
        gidx = tid * G0 + g
        gsb0 = gidx * GRP
        pltpu.sync_copy(src_hbm.at[pl.ds(gsb0, GRP)], srcbuf)
        pltpu.sync_copy(dst_hbm.at[pl.ds(gsb0, GRP)], dstbuf)

        def sb_body(j, carry2):
            @pl.when(gsb0 + j < SB_REAL)
            def _():
                pltpu.async_copy(ad_hbm.at[srcbuf.at[j]], arows, sem).wait()
                pltpu.async_copy(ad_hbm.at[dstbuf.at[j]], brows, sem).wait()
                for ch in range(SB // 16):
                    ridx = lane + (ch * 16)
                    for h in range(H):
                        hcol = jnp.full((16,), h, jnp.int32)
                        av = plsc.load_gather(arows, [ridx, hcol])
                        bv = plsc.load_gather(brows, [ridx, hcol + H])
                        e = av + bv
                        w = jnp.exp(jnp.maximum(e, 0.2 * e))
                        wtbuf[h, j, pl.ds(ch * 16, 16)] = w
                        plsc.store_scatter(wbuf, [ridx, hcol], w)
                pltpu.sync_copy(wbuf, acc_sh.at[dstbuf.at[j]], add=True)
            return carry2

        lax.fori_loop(0, GRP, sb_body, 0)
        for h in range(H):
            pltpu.sync_copy(wtbuf.at[h], wt_hbm.at[h, pl.ds(gsb0, GRP)])
        return carry

    lax.fori_loop(0, G0, group_body, 0)
    plsc.subcore_barrier()
    pltpu.sync_copy(acc_sh.at[pl.ds(s * NPT, NPT)],
                    s_hbm.at[c, pl.ds(s * NPT, NPT)])


@functools.partial(
    pl.kernel,
    out_type=jax.ShapeDtypeStruct((2 * NC, NPAD, 16), jnp.float32),
    mesh=_mesh,
    compiler_params=pltpu.CompilerParams(needs_layout_passes=False, use_tc_tiling_on_sc=False),
    scratch_types=[
        pltpu.VMEM_SHARED((NPAD, 16), jnp.float32),
        pltpu.VMEM((GRP, SB), jnp.int32),
        pltpu.VMEM((GRP, SB), jnp.int32),
        pltpu.VMEM((GRP, SB), jnp.int32),
        pltpu.VMEM((GRP, SB), jnp.float32),
        pltpu.VMEM((GRP, SB), jnp.float32),
        pltpu.VMEM((SB, 16), jnp.float32),
        pltpu.VMEM((SB, 16), jnp.float32),
        pltpu.SemaphoreType.DMA,
    ],
)
def _sc_main(hp_hbm, src_hbm, dst_hbm, wt_hbm, zero_hbm, acc_hbm,
             acc_sh, srcbuf, dstbuf, idxbuf, w0buf, w1buf, hrows, msgbuf, sem):
    c = lax.axis_index("c")
    s = lax.axis_index("s")
    lane = lax.iota(jnp.int32, 16)

    for p_local in range(2):
        pair = c * 2 + p_local
        h0 = 2 * pair
        h1 = h0 + 1
        pltpu.sync_copy(zero_hbm, acc_sh.at[pl.ds(s * NPT, NPT)])
        plsc.subcore_barrier()

        def group_body(g, carry):
            gidx = s * G1 + g
            gsb0 = gidx * GRP
            pltpu.sync_copy(src_hbm.at[pl.ds(gsb0, GRP)], srcbuf)
            pltpu.sync_copy(dst_hbm.at[pl.ds(gsb0, GRP)], dstbuf)
            pltpu.sync_copy(wt_hbm.at[h0, pl.ds(gsb0, GRP)], w0buf)
            pltpu.sync_copy(wt_hbm.at[h1, pl.ds(gsb0, GRP)], w1buf)

            def sb_body(j, carry2):
                @pl.when(gsb0 + j < SB_REAL)
                def _():
                    for ch in range(SB // 16):
                        sv = srcbuf[j, pl.ds(ch * 16, 16)]
                        idxbuf[j, pl.ds(ch * 16, 16)] = sv * 4 + pair
                    pltpu.async_copy(hp_hbm.at[idxbuf.at[j]], hrows, sem).wait()
                    for ch in range(SB // 16):
                        ridx = lane + (ch * 16)
                        w0 = w0buf[j, pl.ds(ch * 16, 16)]
                        w1 = w1buf[j, pl.ds(ch * 16, 16)]
                        for f in range(16):
                            fcol = jnp.full((16,), f, jnp.int32)
                            hv = plsc.load_gather(hrows, [ridx, fcol])
                            m = hv * (w0 if f < 8 else w1)
                            plsc.store_scatter(msgbuf, [ridx, fcol], m)
                    pltpu.sync_copy(msgbuf, acc_sh.at[dstbuf.at[j]], add=True)
                return carry2

            lax.fori_loop(0, GRP, sb_body, 0)
            return carry

        lax.fori_loop(0, G1, group_body, 0)
        plsc.subcore_barrier()
        pltpu.sync_copy(acc_sh.at[pl.ds(s * NPT, NPT)],
                        acc_hbm.at[pair, pl.ds(s * NPT, NPT)])
        plsc.subcore_barrier()


def _combine_body(a0_ref, a1_ref, a2_ref, a3_ref, s0_ref, s1_ref, wl_ref,
                  h_ref, r_ref, b_ref, out_ref):
    s_tot = s0_ref[:, :H] + s1_ref[:, :H] + wl_ref[...]
    r = r_ref[...]
    s64 = jnp.dot(s_tot, r, preferred_element_type=jnp.float32)
    wl64 = jnp.dot(wl_ref[...], r, preferred_element_type=jnp.float32)
    acc = jnp.concatenate(
        [a0_ref[...], a1_ref[...], a2_ref[...], a3_ref[...]], axis=1)
    out_ref[...] = (acc + wl64 * h_ref[...]) / (s64 + 1e-16) + b_ref[...]


def _combine(a0, a1, a2, a3, s0, s1, wl, h, R, bias2d):
    blk = 1000
    node_spec16 = pl.BlockSpec((blk, 16), lambda i: (i, 0))
    return pl.pallas_call(
        _combine_body,
        grid=(N // blk,),
        in_specs=[
            node_spec16, node_spec16, node_spec16, node_spec16,
            node_spec16, node_spec16,
            pl.BlockSpec((blk, H), lambda i: (i, 0)),
            pl.BlockSpec((blk, HF), lambda i: (i, 0)),
            pl.BlockSpec((H, HF), lambda i: (0, 0)),
            pl.BlockSpec((1, HF), lambda i: (0, 0)),
        ],
        out_specs=pl.BlockSpec((blk, HF), lambda i: (i, 0)),
        out_shape=jax.ShapeDtypeStruct((N, HF), jnp.float32),
    )(a0, a1, a2, a3, s0, s1, wl, h, R, bias2d)


def kernel(x, edge_index, W, att_src, att_dst, bias):
    f32 = jnp.float32
    a_s = att_src.reshape(H, F).astype(f32)
    a_d = att_dst.reshape(H, F).astype(f32)
    eye = jnp.eye(H, dtype=f32)
    Ms = (a_s[:, :, None] * eye[:, None, :]).reshape(HF, H)
    Md = (a_d[:, :, None] * eye[:, None, :]).reshape(HF, H)
    Mcat = jnp.concatenate([Ms, Md], axis=1)
    R = jnp.repeat(eye, F, axis=1)

    pad = E_PAD - E
    src_p = jnp.concatenate(
        [edge_index[0], jnp.zeros((pad,), jnp.int32)]).reshape(-1, SB)
    dst_p = jnp.concatenate(
        [edge_index[1], jnp.zeros((pad,), jnp.int32)]).reshape(-1, SB)

    h, ad, wl = _dense_prologue(x.astype(f32), W.astype(f32), Mcat)
    zeros = jnp.zeros((NPT, 16), f32)

    wt, s_part = _sc_pass0(ad, src_p, dst_p, zeros)
    hp = h.reshape(N * 4, 16)
    acc = _sc_main(hp, src_p, dst_p, wt, zeros)

    out = _combine(acc[0], acc[1], acc[2], acc[3],
                   s_part[0], s_part[1], wl, h, R,
                   bias.astype(f32).reshape(1, HF))
    return out


# speedup vs baseline: 66.9088x; 1.4587x over previous
import functools

import jax
import jax.numpy as jnp
from jax import lax
from jax.experimental import pallas as pl
from jax.experimental.pallas import tpu as pltpu
from jax.experimental.pallas import tpu_sc as plsc

N = 100000
E = 1600000
D_IN = 34
H = 8
F = 8
HF = H * F

NC, NS = 2, 16
NT = NC * NS
SB = 128
E_PAD = NT * 49 * 1024
SB_REAL = E // SB
NSB = E_PAD // SB
NPT = 6256
NPAD = NS * NPT

GRP1 = 4
NG1 = E_PAD // (NT * GRP1 * SB)
GRP2 = 2
NG2 = E_PAD // (NS * GRP2 * SB)

_mesh = plsc.VectorSubcoreMesh(core_axis_name="c", subcore_axis_name="s",
                               num_cores=NC, num_subcores=NS)
_sc_params = pltpu.CompilerParams(needs_layout_passes=False,
                                  use_tc_tiling_on_sc=False)


def _dense_body(x_ref, w_ref, m_ref, h_ref, ad_ref, wl_ref):
    h = jnp.dot(x_ref[...], w_ref[...], preferred_element_type=jnp.float32)
    h_ref[...] = h
    ad = jnp.dot(h, m_ref[...], preferred_element_type=jnp.float32)
    ad_ref[...] = ad
    e = ad[:, :H] + ad[:, H:]
    wl_ref[...] = jnp.exp(jnp.maximum(e, 0.2 * e))


def _dense_prologue(x, W, Mcat):
    blk = 1000
    return pl.pallas_call(
        _dense_body,
        grid=(N // blk,),
        in_specs=[
            pl.BlockSpec((blk, D_IN), lambda i: (i, 0)),
            pl.BlockSpec((D_IN, HF), lambda i: (0, 0)),
            pl.BlockSpec((HF, 2 * H), lambda i: (0, 0)),
        ],
        out_specs=[
            pl.BlockSpec((blk, HF), lambda i: (i, 0)),
            pl.BlockSpec((blk, 2 * H), lambda i: (i, 0)),
            pl.BlockSpec((blk, H), lambda i: (i, 0)),
        ],
        out_shape=[
            jax.ShapeDtypeStruct((N, HF), jnp.float32),
            jax.ShapeDtypeStruct((N, 2 * H), jnp.float32),
            jax.ShapeDtypeStruct((N, H), jnp.float32),
        ],
    )(x, W, Mcat)


GRP1S = 8
NG1S = E_PAD // (NT * GRP1S * SB)


@functools.partial(
    pl.kernel,
    out_type=[
        jax.ShapeDtypeStruct((H, E_PAD), jnp.float32),
        jax.ShapeDtypeStruct((NC, NPAD, 16), jnp.float32),
    ],
    mesh=_mesh,
    compiler_params=_sc_params,
    scratch_types=[
        pltpu.VMEM_SHARED((NPAD, 16), jnp.float32),
        pltpu.VMEM((GRP1S, SB), jnp.int32),
        pltpu.VMEM((GRP1S, SB), jnp.int32),
        pltpu.VMEM((SB, 16), jnp.float32),
        pltpu.VMEM((SB, 16), jnp.float32),
        pltpu.VMEM((SB, 16), jnp.float32),
        pltpu.VMEM((H, GRP1S * SB), jnp.float32),
        pltpu.SemaphoreType.DMA,
    ],
)
def _sc_pass0(ad_hbm, src_hbm, dst_hbm, zero16_hbm, wt_hbm, s_hbm,
              acc_sh, srcbuf, dstbuf, arows, brows, wbuf, wtbuf, sem):
    c = lax.axis_index("c")
    s = lax.axis_index("s")
    tid = c * NS + s
    EG = GRP1S * SB
    pltpu.sync_copy(zero16_hbm, acc_sh.at[pl.ds(s * NPT, NPT)])
    pltpu.sync_copy(zero16_hbm.at[pl.ds(0, SB)], wbuf)
    plsc.subcore_barrier()

    lane = lax.iota(jnp.int32, 16)

    def group_body(g, carry):
        gidx = tid * NG1S + g
        gsb0 = gidx * GRP1S
        goff = gsb0 * SB
        pltpu.sync_copy(src_hbm.at[pl.ds(gsb0, GRP1S)], srcbuf)
        pltpu.sync_copy(dst_hbm.at[pl.ds(gsb0, GRP1S)], dstbuf)

        def sb_body(j, carry2):
            @pl.when(gsb0 + j < SB_REAL)
            def _():
                pltpu.async_copy(ad_hbm.at[srcbuf.at[j]], arows, sem).wait()
                pltpu.async_copy(ad_hbm.at[dstbuf.at[j]], brows, sem).wait()
                for ch in range(SB // 16):
                    ridx = lane + (ch * 16)
                    for h in range(H):
                        hcol = jnp.full((16,), h, jnp.int32)
                        av = plsc.load_gather(arows, [ridx, hcol])
                        bv = plsc.load_gather(brows, [ridx, hcol + H])
                        e = av + bv
                        w = jnp.exp(jnp.maximum(e, 0.2 * e))
                        wtbuf[h, pl.ds(j * SB + ch * 16, 16)] = w
                        plsc.store_scatter(wbuf, [ridx, hcol], w)
                pltpu.sync_copy(wbuf, acc_sh.at[dstbuf.at[j]], add=True)
            return carry2

        lax.fori_loop(0, GRP1S, sb_body, 0)
        for h in range(H):
            pltpu.sync_copy(wtbuf.at[h], wt_hbm.at[h, pl.ds(goff, EG)])
        return carry

    lax.fori_loop(0, NG1S, group_body, 0)
    plsc.subcore_barrier()
    pltpu.sync_copy(acc_sh.at[pl.ds(s * NPT, NPT)],
                    s_hbm.at[c, pl.ds(s * NPT, NPT)])


@functools.partial(
    pl.kernel,
    out_type=jax.ShapeDtypeStruct((2 * NC, NPAD, 16), jnp.float32),
    mesh=_mesh,
    compiler_params=_sc_params,
    scratch_types=[
        pltpu.VMEM_SHARED((NPAD, 16), jnp.float32),
        pltpu.VMEM((4, GRP2, SB), jnp.int32),
        pltpu.VMEM((4, GRP2, SB), jnp.int32),
        pltpu.VMEM((4, GRP2 * SB), jnp.float32),
        pltpu.VMEM((4, GRP2 * SB), jnp.float32),
        pltpu.VMEM((2, GRP2 * SB, 16), jnp.float32),
        pltpu.VMEM((2, GRP2 * SB, 16), jnp.float32),
        pltpu.SemaphoreType.DMA, pltpu.SemaphoreType.DMA,
        pltpu.SemaphoreType.DMA, pltpu.SemaphoreType.DMA,
        pltpu.SemaphoreType.DMA, pltpu.SemaphoreType.DMA,
        pltpu.SemaphoreType.DMA, pltpu.SemaphoreType.DMA,
    ],
)
def _sc_main(hp_hbm, src_hbm, dst_hbm, wt_hbm, zero16_hbm, acc_hbm,
             acc_sh, srcb, dstb, w0b, w1b, hrows, msgb,
             l0, l1, l2, l3, ga, gb, sa, sb_):
    c = lax.axis_index("c")
    s = lax.axis_index("s")
    semL = [l0, l1, l2, l3]
    semG = [ga, gb]
    semS = [sa, sb_]
    lane = lax.iota(jnp.int32, 16)
    EG = GRP2 * SB

    def gsb0_of(g):
        return (s * NG2 + g) * GRP2

    for p_local in range(2):
        pair = c * 2 + p_local
        h0 = 2 * pair
        h1 = h0 + 1

        pltpu.sync_copy(zero16_hbm, acc_sh.at[pl.ds(s * NPT, NPT)])
        plsc.subcore_barrier()

        def issue_lin(g, d4):
            gsb0 = gsb0_of(g)
            goff = gsb0 * SB
            pltpu.async_copy(src_hbm.at[pl.ds(gsb0, GRP2)], srcb.at[d4],
                             semL[d4])
            pltpu.async_copy(dst_hbm.at[pl.ds(gsb0, GRP2)], dstb.at[d4],
                             semL[d4])
            pltpu.async_copy(wt_hbm.at[h0, pl.ds(goff, EG)], w0b.at[d4],
                             semL[d4])
            pltpu.async_copy(wt_hbm.at[h1, pl.ds(goff, EG)], w1b.at[d4],
                             semL[d4])

        def drain_lin(d4):
            for _ in range(2):
                pltpu.make_async_copy(src_hbm.at[pl.ds(0, GRP2)], srcb.at[d4],
                                      semL[d4]).wait()
            for _ in range(2):
                pltpu.make_async_copy(wt_hbm.at[0, pl.ds(0, EG)], w0b.at[d4],
                                      semL[d4]).wait()

        def compute_idx(d4):
            def j_body(j, carry):
                for ch in range(SB // 16):
                    sv = srcb[d4, j, pl.ds(ch * 16, 16)]
                    srcb[d4, j, pl.ds(ch * 16, 16)] = sv * 4 + pair
                return carry
            lax.fori_loop(0, GRP2, j_body, 0)

        def issue_gathers(d4, d2):
            def j_body(j, carry):
                pltpu.async_copy(hp_hbm.at[srcb.at[d4, j]],
                                 hrows.at[d2, pl.ds(j * SB, SB)], semG[d2])
                return carry
            lax.fori_loop(0, GRP2, j_body, 0)

        def drain_gathers(d2):
            for j in range(GRP2):
                pltpu.make_async_copy(hp_hbm.at[srcb.at[0, j]],
                                      hrows.at[d2, pl.ds(j * SB, SB)],
                                      semG[d2]).wait()

        def compute(g, d4, d2):
            gsb0 = gsb0_of(g)

            def j_body(j, carry):
                realf = jnp.where(gsb0 + j < SB_REAL, 1.0,
                                  0.0).astype(jnp.float32)
                for ch in range(SB // 16):
                    ridx = lane + (j * SB + ch * 16)
                    w0v = w0b[d4, pl.ds(j * SB + ch * 16, 16)] * realf
                    w1v = w1b[d4, pl.ds(j * SB + ch * 16, 16)] * realf
                    for f in range(16):
                        fcol = jnp.full((16,), f, jnp.int32)
                        hv = plsc.load_gather(hrows.at[d2], [ridx, fcol])
                        m = hv * (w0v if f < 8 else w1v)
                        plsc.store_scatter(msgb.at[d2], [ridx, fcol], m)
                return carry

            lax.fori_loop(0, GRP2, j_body, 0)

        def issue_scatters(d4, d2):
            def j_body(j, carry):
                pltpu.async_copy(msgb.at[d2, pl.ds(j * SB, SB)],
                                 acc_sh.at[dstb.at[d4, j]], semS[d2], add=True)
                return carry
            lax.fori_loop(0, GRP2, j_body, 0)

        def drain_scatters(d2):
            for j in range(GRP2):
                pltpu.make_async_copy(msgb.at[d2, pl.ds(j * SB, SB)],
                                      acc_sh.at[dstb.at[0, j]],
                                      semS[d2]).wait()

        issue_lin(0, 0)
        issue_lin(1, 1)
        drain_lin(0)
        compute_idx(0)
        issue_gathers(0, 0)

        n_outer = (NG2 + 2 + 3) // 4

        def outer(o, carry):
            for db in range(4):
                g = o * 4 + db
                d2, d4 = db % 2, db
                d2n, d4n = (db + 1) % 2, (db + 1) % 4
                d4nn = (db + 2) % 4

                @pl.when(jnp.logical_and(g >= 2, g < NG2 + 2))
                def _():
                    drain_scatters(d2)

                @pl.when(g + 2 < NG2)
                def _():
                    issue_lin(g + 2, d4nn)

                @pl.when(g < NG2)
                def _():
                    drain_gathers(d2)

                @pl.when(g + 1 < NG2)
                def _():
                    drain_lin(d4n)
                    compute_idx(d4n)
                    issue_gathers(d4n, d2n)

                @pl.when(g < NG2)
                def _():
                    compute(g, d4, d2)
                    issue_scatters(d4, d2)
            return carry

        lax.fori_loop(0, n_outer, outer, 0)

        plsc.subcore_barrier()
        pltpu.sync_copy(acc_sh.at[pl.ds(s * NPT, NPT)],
                        acc_hbm.at[pair, pl.ds(s * NPT, NPT)])
        plsc.subcore_barrier()


def _combine_body(a0_ref, a1_ref, a2_ref, a3_ref, s0_ref, s1_ref, wl_ref,
                  h_ref, r_ref, b_ref, out_ref):
    s_tot = s0_ref[:, :H] + s1_ref[:, :H] + wl_ref[...]
    r = r_ref[...]
    s64 = jnp.dot(s_tot, r, preferred_element_type=jnp.float32)
    wl64 = jnp.dot(wl_ref[...], r, preferred_element_type=jnp.float32)
    acc = jnp.concatenate(
        [a0_ref[...], a1_ref[...], a2_ref[...], a3_ref[...]], axis=1)
    out_ref[...] = (acc + wl64 * h_ref[...]) / (s64 + 1e-16) + b_ref[...]


def _combine(a0, a1, a2, a3, s0, s1, wl, h, R, bias2d):
    blk = 1000
    node16 = pl.BlockSpec((blk, 16), lambda i: (i, 0))
    node8 = pl.BlockSpec((blk, H), lambda i: (i, 0))
    return pl.pallas_call(
        _combine_body,
        grid=(N // blk,),
        in_specs=[
            node16, node16, node16, node16,
            node16, node16, node8,
            pl.BlockSpec((blk, HF), lambda i: (i, 0)),
            pl.BlockSpec((H, HF), lambda i: (0, 0)),
            pl.BlockSpec((1, HF), lambda i: (0, 0)),
        ],
        out_specs=pl.BlockSpec((blk, HF), lambda i: (i, 0)),
        out_shape=jax.ShapeDtypeStruct((N, HF), jnp.float32),
    )(a0, a1, a2, a3, s0, s1, wl, h, R, bias2d)


def kernel(x, edge_index, W, att_src, att_dst, bias):
    f32 = jnp.float32
    a_s = att_src.reshape(H, F).astype(f32)
    a_d = att_dst.reshape(H, F).astype(f32)
    eye = jnp.eye(H, dtype=f32)
    Ms = (a_s[:, :, None] * eye[:, None, :]).reshape(HF, H)
    Md = (a_d[:, :, None] * eye[:, None, :]).reshape(HF, H)
    Mcat = jnp.concatenate([Ms, Md], axis=1)
    R = jnp.repeat(eye, F, axis=1)

    pad = E_PAD - E
    src_p = jnp.concatenate(
        [edge_index[0], jnp.zeros((pad,), jnp.int32)]).reshape(-1, SB)
    dst_p = jnp.concatenate(
        [edge_index[1], jnp.zeros((pad,), jnp.int32)]).reshape(-1, SB)

    h, ad, wl = _dense_prologue(x.astype(f32), W.astype(f32), Mcat)
    zeros16 = jnp.zeros((NPT, 16), f32)

    wt, s_part = _sc_pass0(ad, src_p, dst_p, zeros16)
    hp = h.reshape(N * 4, 16)
    acc = _sc_main(hp, src_p, dst_p, wt, zeros16)

    out = _combine(acc[0], acc[1], acc[2], acc[3],
                   s_part[0], s_part[1], wl, h, R,
                   bias.astype(f32).reshape(1, HF))
    return out


# speedup vs baseline: 80.5547x; 1.2039x over previous
import functools

import jax
import jax.numpy as jnp
from jax import lax
from jax.experimental import pallas as pl
from jax.experimental.pallas import tpu as pltpu
from jax.experimental.pallas import tpu_sc as plsc

N = 100000
E = 1600000
D_IN = 34
H = 8
F = 8
HF = H * F

NC, NS = 2, 16
NT = NC * NS
SB = 128
E_PAD = NT * 49 * 1024
SB_REAL = E // SB
NSB = E_PAD // SB
NPT = 6256
NPAD = NS * NPT

GRP1 = 4
NG1 = E_PAD // (NT * GRP1 * SB)
GRP2 = 2
NG2 = E_PAD // (NS * GRP2 * SB)

_mesh = plsc.VectorSubcoreMesh(core_axis_name="c", subcore_axis_name="s",
                               num_cores=NC, num_subcores=NS)
_sc_params = pltpu.CompilerParams(needs_layout_passes=False,
                                  use_tc_tiling_on_sc=False)


def _dense_body(x_ref, w_ref, m_ref, h_ref, ad_ref, wl_ref):
    h = jnp.dot(x_ref[...], w_ref[...], preferred_element_type=jnp.float32)
    h_ref[...] = h
    ad = jnp.dot(h, m_ref[...], preferred_element_type=jnp.float32)
    ad_ref[...] = ad
    e = ad[:, :H] + ad[:, H:]
    wl_ref[...] = jnp.exp(jnp.maximum(e, 0.2 * e))


def _dense_prologue(x, W, Mcat):
    blk = 1000
    return pl.pallas_call(
        _dense_body,
        grid=(N // blk,),
        in_specs=[
            pl.BlockSpec((blk, D_IN), lambda i: (i, 0)),
            pl.BlockSpec((D_IN, HF), lambda i: (0, 0)),
            pl.BlockSpec((HF, 2 * H), lambda i: (0, 0)),
        ],
        out_specs=[
            pl.BlockSpec((blk, HF), lambda i: (i, 0)),
            pl.BlockSpec((blk, 2 * H), lambda i: (i, 0)),
            pl.BlockSpec((blk, H), lambda i: (i, 0)),
        ],
        out_shape=[
            jax.ShapeDtypeStruct((N, HF), jnp.float32),
            jax.ShapeDtypeStruct((N, 2 * H), jnp.float32),
            jax.ShapeDtypeStruct((N, H), jnp.float32),
        ],
    )(x, W, Mcat)


GRP1 = 2
NG1 = E_PAD // (NT * GRP1 * SB)


@functools.partial(
    pl.kernel,
    out_type=[
        jax.ShapeDtypeStruct((H, E_PAD), jnp.float32),
        jax.ShapeDtypeStruct((NC, NPAD, 16), jnp.float32),
    ],
    mesh=_mesh,
    compiler_params=_sc_params,
    scratch_types=[
        pltpu.VMEM_SHARED((NPAD, 16), jnp.float32),
        pltpu.VMEM((4, GRP1, SB), jnp.int32),
        pltpu.VMEM((4, GRP1, SB), jnp.int32),
        pltpu.VMEM((2, GRP1 * SB, 16), jnp.float32),
        pltpu.VMEM((2, GRP1 * SB, 16), jnp.float32),
        pltpu.VMEM((2, GRP1 * SB, 16), jnp.float32),
        pltpu.VMEM((2, H, GRP1 * SB), jnp.float32),
        pltpu.SemaphoreType.DMA, pltpu.SemaphoreType.DMA,
        pltpu.SemaphoreType.DMA, pltpu.SemaphoreType.DMA,
        pltpu.SemaphoreType.DMA, pltpu.SemaphoreType.DMA,
        pltpu.SemaphoreType.DMA, pltpu.SemaphoreType.DMA,
        pltpu.SemaphoreType.DMA, pltpu.SemaphoreType.DMA,
    ],
)
def _sc_pass0(ad_hbm, src_hbm, dst_hbm, zero16_hbm, wt_hbm, s_hbm,
              s_sh, srcb, dstb, arows, brows, wgrp, wtb,
              l0, l1, l2, l3, ga, gb, sa, sb_, wa, wb):
    c = lax.axis_index("c")
    s = lax.axis_index("s")
    tid = c * NS + s
    semL = [l0, l1, l2, l3]
    semG = [ga, gb]
    semS = [sa, sb_]
    semW = [wa, wb]
    lane = lax.iota(jnp.int32, 16)
    EG = GRP1 * SB

    pltpu.sync_copy(zero16_hbm, s_sh.at[pl.ds(s * NPT, NPT)])
    pltpu.sync_copy(zero16_hbm.at[pl.ds(0, EG)], wgrp.at[0])
    pltpu.sync_copy(zero16_hbm.at[pl.ds(0, EG)], wgrp.at[1])
    plsc.subcore_barrier()

    def gsb0_of(g):
        return (tid * NG1 + g) * GRP1

    def issue_lin(g, d4):
        pltpu.async_copy(src_hbm.at[pl.ds(gsb0_of(g), GRP1)], srcb.at[d4],
                         semL[d4])
        pltpu.async_copy(dst_hbm.at[pl.ds(gsb0_of(g), GRP1)], dstb.at[d4],
                         semL[d4])

    def drain_lin(d4):
        pltpu.make_async_copy(src_hbm.at[pl.ds(0, GRP1)], srcb.at[d4],
                              semL[d4]).wait()
        pltpu.make_async_copy(dst_hbm.at[pl.ds(0, GRP1)], dstb.at[d4],
                              semL[d4]).wait()

    def issue_gathers(d4, d2):
        def j_body(j, carry):
            pltpu.async_copy(ad_hbm.at[srcb.at[d4, j]],
                             arows.at[d2, pl.ds(j * SB, SB)], semG[d2])
            pltpu.async_copy(ad_hbm.at[dstb.at[d4, j]],
                             brows.at[d2, pl.ds(j * SB, SB)], semG[d2])
            return carry
        lax.fori_loop(0, GRP1, j_body, 0)

    def drain_gathers(d2):
        for j in range(GRP1):
            pltpu.make_async_copy(ad_hbm.at[srcb.at[0, j]],
                                  arows.at[d2, pl.ds(j * SB, SB)],
                                  semG[d2]).wait()
            pltpu.make_async_copy(ad_hbm.at[dstb.at[0, j]],
                                  brows.at[d2, pl.ds(j * SB, SB)],
                                  semG[d2]).wait()

    def compute(g, d2):
        gsb0 = gsb0_of(g)

        def j_body(j, carry):
            realf = jnp.where(gsb0 + j < SB_REAL, 1.0, 0.0).astype(jnp.float32)
            for ch in range(SB // 16):
                ridx = lane + (j * SB + ch * 16)
                for h in range(H):
                    hcol = jnp.full((16,), h, jnp.int32)
                    av = plsc.load_gather(arows.at[d2], [ridx, hcol])
                    bv = plsc.load_gather(brows.at[d2], [ridx, hcol + H])
                    e = av + bv
                    w = jnp.exp(jnp.maximum(e, 0.2 * e))
                    wtb[d2, h, pl.ds(j * SB + ch * 16, 16)] = w
                    plsc.store_scatter(wgrp.at[d2], [ridx, hcol], w * realf)
            return carry

        lax.fori_loop(0, GRP1, j_body, 0)

    def issue_out(g, d4, d2):
        goff = gsb0_of(g) * SB

        def j_body(j, carry):
            pltpu.async_copy(wgrp.at[d2, pl.ds(j * SB, SB)],
                             s_sh.at[dstb.at[d4, j]], semS[d2], add=True)
            return carry
        lax.fori_loop(0, GRP1, j_body, 0)
        for h in range(H):
            pltpu.async_copy(wtb.at[d2, h], wt_hbm.at[h, pl.ds(goff, EG)],
                             semW[d2])

    def drain_out(d2):
        for j in range(GRP1):
            pltpu.make_async_copy(wgrp.at[d2, pl.ds(j * SB, SB)],
                                  s_sh.at[dstb.at[0, j]], semS[d2]).wait()
        for h in range(H):
            pltpu.make_async_copy(wtb.at[d2, h],
                                  wt_hbm.at[h, pl.ds(0, EG)], semW[d2]).wait()

    issue_lin(0, 0)
    issue_lin(1, 1)
    drain_lin(0)
    issue_gathers(0, 0)

    n_outer = (NG1 + 2 + 3) // 4

    def outer(o, carry):
        for db in range(4):
            g = o * 4 + db
            d2, d4 = db % 2, db
            d2n, d4n = (db + 1) % 2, (db + 1) % 4
            d4nn = (db + 2) % 4

            @pl.when(jnp.logical_and(g >= 2, g < NG1 + 2))
            def _():
                drain_out(d2)

            @pl.when(g + 2 < NG1)
            def _():
                issue_lin(g + 2, d4nn)

            @pl.when(g < NG1)
            def _():
                drain_gathers(d2)

            @pl.when(g + 1 < NG1)
            def _():
                drain_lin(d4n)
                issue_gathers(d4n, d2n)

            @pl.when(g < NG1)
            def _():
                compute(g, d2)
                issue_out(g, d4, d2)
        return carry

    lax.fori_loop(0, n_outer, outer, 0)

    plsc.subcore_barrier()
    pltpu.sync_copy(s_sh.at[pl.ds(s * NPT, NPT)],
                    s_hbm.at[c, pl.ds(s * NPT, NPT)])


@functools.partial(
    pl.kernel,
    out_type=jax.ShapeDtypeStruct((2 * NC, NPAD, 16), jnp.float32),
    mesh=_mesh,
    compiler_params=_sc_params,
    scratch_types=[
        pltpu.VMEM_SHARED((NPAD, 16), jnp.float32),
        pltpu.VMEM((4, GRP2, SB), jnp.int32),
        pltpu.VMEM((4, GRP2, SB), jnp.int32),
        pltpu.VMEM((4, GRP2 * SB), jnp.float32),
        pltpu.VMEM((4, GRP2 * SB), jnp.float32),
        pltpu.VMEM((2, GRP2 * SB, 16), jnp.float32),
        pltpu.VMEM((2, GRP2 * SB, 16), jnp.float32),
        pltpu.SemaphoreType.DMA, pltpu.SemaphoreType.DMA,
        pltpu.SemaphoreType.DMA, pltpu.SemaphoreType.DMA,
        pltpu.SemaphoreType.DMA, pltpu.SemaphoreType.DMA,
        pltpu.SemaphoreType.DMA, pltpu.SemaphoreType.DMA,
    ],
)
def _sc_main(hp_hbm, src_hbm, dst_hbm, wt_hbm, zero16_hbm, acc_hbm,
             acc_sh, srcb, dstb, w0b, w1b, hrows, msgb,
             l0, l1, l2, l3, ga, gb, sa, sb_):
    c = lax.axis_index("c")
    s = lax.axis_index("s")
    semL = [l0, l1, l2, l3]
    semG = [ga, gb]
    semS = [sa, sb_]
    lane = lax.iota(jnp.int32, 16)
    EG = GRP2 * SB

    def gsb0_of(g):
        return (s * NG2 + g) * GRP2

    for p_local in range(2):
        pair = c * 2 + p_local
        h0 = 2 * pair
        h1 = h0 + 1

        pltpu.sync_copy(zero16_hbm, acc_sh.at[pl.ds(s * NPT, NPT)])
        plsc.subcore_barrier()

        def issue_lin(g, d4):
            gsb0 = gsb0_of(g)
            goff = gsb0 * SB
            pltpu.async_copy(src_hbm.at[pl.ds(gsb0, GRP2)], srcb.at[d4],
                             semL[d4])
            pltpu.async_copy(dst_hbm.at[pl.ds(gsb0, GRP2)], dstb.at[d4],
                             semL[d4])
            pltpu.async_copy(wt_hbm.at[h0, pl.ds(goff, EG)], w0b.at[d4],
                             semL[d4])
            pltpu.async_copy(wt_hbm.at[h1, pl.ds(goff, EG)], w1b.at[d4],
                             semL[d4])

        def drain_lin(d4):
            for _ in range(2):
                pltpu.make_async_copy(src_hbm.at[pl.ds(0, GRP2)], srcb.at[d4],
                                      semL[d4]).wait()
            for _ in range(2):
                pltpu.make_async_copy(wt_hbm.at[0, pl.ds(0, EG)], w0b.at[d4],
                                      semL[d4]).wait()

        def compute_idx(d4):
            def j_body(j, carry):
                for ch in range(SB // 16):
                    sv = srcb[d4, j, pl.ds(ch * 16, 16)]
                    srcb[d4, j, pl.ds(ch * 16, 16)] = sv * 4 + pair
                return carry
            lax.fori_loop(0, GRP2, j_body, 0)

        def issue_gathers(d4, d2):
            def j_body(j, carry):
                pltpu.async_copy(hp_hbm.at[srcb.at[d4, j]],
                                 hrows.at[d2, pl.ds(j * SB, SB)], semG[d2])
                return carry
            lax.fori_loop(0, GRP2, j_body, 0)

        def drain_gathers(d2):
            for j in range(GRP2):
                pltpu.make_async_copy(hp_hbm.at[srcb.at[0, j]],
                                      hrows.at[d2, pl.ds(j * SB, SB)],
                                      semG[d2]).wait()

        def compute(g, d4, d2):
            gsb0 = gsb0_of(g)

            def j_body(j, carry):
                realf = jnp.where(gsb0 + j < SB_REAL, 1.0,
                                  0.0).astype(jnp.float32)
                for ch in range(SB // 16):
                    ridx = lane + (j * SB + ch * 16)
                    w0v = w0b[d4, pl.ds(j * SB + ch * 16, 16)] * realf
                    w1v = w1b[d4, pl.ds(j * SB + ch * 16, 16)] * realf
                    for f in range(16):
                        fcol = jnp.full((16,), f, jnp.int32)
                        hv = plsc.load_gather(hrows.at[d2], [ridx, fcol])
                        m = hv * (w0v if f < 8 else w1v)
                        plsc.store_scatter(msgb.at[d2], [ridx, fcol], m)
                return carry

            lax.fori_loop(0, GRP2, j_body, 0)

        def issue_scatters(d4, d2):
            def j_body(j, carry):
                pltpu.async_copy(msgb.at[d2, pl.ds(j * SB, SB)],
                                 acc_sh.at[dstb.at[d4, j]], semS[d2], add=True)
                return carry
            lax.fori_loop(0, GRP2, j_body, 0)

        def drain_scatters(d2):
            for j in range(GRP2):
                pltpu.make_async_copy(msgb.at[d2, pl.ds(j * SB, SB)],
                                      acc_sh.at[dstb.at[0, j]],
                                      semS[d2]).wait()

        issue_lin(0, 0)
        issue_lin(1, 1)
        drain_lin(0)
        compute_idx(0)
        issue_gathers(0, 0)

        n_outer = (NG2 + 2 + 3) // 4

        def outer(o, carry):
            for db in range(4):
                g = o * 4 + db
                d2, d4 = db % 2, db
                d2n, d4n = (db + 1) % 2, (db + 1) % 4
                d4nn = (db + 2) % 4

                @pl.when(jnp.logical_and(g >= 2, g < NG2 + 2))
                def _():
                    drain_scatters(d2)

                @pl.when(g + 2 < NG2)
                def _():
                    issue_lin(g + 2, d4nn)

                @pl.when(g < NG2)
                def _():
                    drain_gathers(d2)

                @pl.when(g + 1 < NG2)
                def _():
                    drain_lin(d4n)
                    compute_idx(d4n)
                    issue_gathers(d4n, d2n)

                @pl.when(g < NG2)
                def _():
                    compute(g, d4, d2)
                    issue_scatters(d4, d2)
            return carry

        lax.fori_loop(0, n_outer, outer, 0)

        plsc.subcore_barrier()
        pltpu.sync_copy(acc_sh.at[pl.ds(s * NPT, NPT)],
                        acc_hbm.at[pair, pl.ds(s * NPT, NPT)])
        plsc.subcore_barrier()


def _combine_body(a0_ref, a1_ref, a2_ref, a3_ref, s0_ref, s1_ref, wl_ref,
                  h_ref, r_ref, b_ref, out_ref):
    s_tot = s0_ref[:, :H] + s1_ref[:, :H] + wl_ref[...]
    r = r_ref[...]
    s64 = jnp.dot(s_tot, r, preferred_element_type=jnp.float32)
    wl64 = jnp.dot(wl_ref[...], r, preferred_element_type=jnp.float32)
    acc = jnp.concatenate(
        [a0_ref[...], a1_ref[...], a2_ref[...], a3_ref[...]], axis=1)
    out_ref[...] = (acc + wl64 * h_ref[...]) / (s64 + 1e-16) + b_ref[...]


def _combine(a0, a1, a2, a3, s0, s1, wl, h, R, bias2d):
    blk = 1000
    node16 = pl.BlockSpec((blk, 16), lambda i: (i, 0))
    node8 = pl.BlockSpec((blk, H), lambda i: (i, 0))
    return pl.pallas_call(
        _combine_body,
        grid=(N // blk,),
        in_specs=[
            node16, node16, node16, node16,
            node16, node16, node8,
            pl.BlockSpec((blk, HF), lambda i: (i, 0)),
            pl.BlockSpec((H, HF), lambda i: (0, 0)),
            pl.BlockSpec((1, HF), lambda i: (0, 0)),
        ],
        out_specs=pl.BlockSpec((blk, HF), lambda i: (i, 0)),
        out_shape=jax.ShapeDtypeStruct((N, HF), jnp.float32),
    )(a0, a1, a2, a3, s0, s1, wl, h, R, bias2d)


def kernel(x, edge_index, W, att_src, att_dst, bias):
    f32 = jnp.float32
    a_s = att_src.reshape(H, F).astype(f32)
    a_d = att_dst.reshape(H, F).astype(f32)
    eye = jnp.eye(H, dtype=f32)
    Ms = (a_s[:, :, None] * eye[:, None, :]).reshape(HF, H)
    Md = (a_d[:, :, None] * eye[:, None, :]).reshape(HF, H)
    Mcat = jnp.concatenate([Ms, Md], axis=1)
    R = jnp.repeat(eye, F, axis=1)

    pad = E_PAD - E
    src_p = jnp.concatenate(
        [edge_index[0], jnp.zeros((pad,), jnp.int32)]).reshape(-1, SB)
    dst_p = jnp.concatenate(
        [edge_index[1], jnp.zeros((pad,), jnp.int32)]).reshape(-1, SB)

    h, ad, wl = _dense_prologue(x.astype(f32), W.astype(f32), Mcat)
    zeros16 = jnp.zeros((NPT, 16), f32)

    wt, s_part = _sc_pass0(ad, src_p, dst_p, zeros16)
    hp = h.reshape(N * 4, 16)
    acc = _sc_main(hp, src_p, dst_p, wt, zeros16)

    out = _combine(acc[0], acc[1], acc[2], acc[3],
                   s_part[0], s_part[1], wl, h, R,
                   bias.astype(f32).reshape(1, HF))
    return out


# speedup vs baseline: 82.3406x; 1.0222x over previous
import functools

import jax
import jax.numpy as jnp
from jax import lax
from jax.experimental import pallas as pl
from jax.experimental.pallas import tpu as pltpu
from jax.experimental.pallas import tpu_sc as plsc

N = 100000
E = 1600000
D_IN = 34
H = 8
F = 8
HF = H * F

NC, NS = 2, 16
NT = NC * NS
SB = 128
E_PAD = NT * 49 * 1024
SB_REAL = E // SB
NSB = E_PAD // SB
NPT = 6256
NPAD = NS * NPT

GRP1 = 4
NG1 = E_PAD // (NT * GRP1 * SB)
GRP2 = 2
NG2 = E_PAD // (NS * GRP2 * SB)

_mesh = plsc.VectorSubcoreMesh(core_axis_name="c", subcore_axis_name="s",
                               num_cores=NC, num_subcores=NS)
_sc_params = pltpu.CompilerParams(needs_layout_passes=False,
                                  use_tc_tiling_on_sc=False)


def _dense_body(x_ref, w_ref, m_ref, h_ref, ad_ref, wl_ref):
    h = jnp.dot(x_ref[...], w_ref[...], preferred_element_type=jnp.float32)
    h_ref[...] = h
    ad = jnp.dot(h, m_ref[...], preferred_element_type=jnp.float32)
    ad_ref[...] = ad
    e = ad[:, :H] + ad[:, H:]
    wl_ref[...] = jnp.exp(jnp.maximum(e, 0.2 * e))


def _dense_prologue(x, W, Mcat):
    blk = 1000
    return pl.pallas_call(
        _dense_body,
        grid=(N // blk,),
        in_specs=[
            pl.BlockSpec((blk, D_IN), lambda i: (i, 0)),
            pl.BlockSpec((D_IN, HF), lambda i: (0, 0)),
            pl.BlockSpec((HF, 2 * H), lambda i: (0, 0)),
        ],
        out_specs=[
            pl.BlockSpec((blk, HF), lambda i: (i, 0)),
            pl.BlockSpec((blk, 2 * H), lambda i: (i, 0)),
            pl.BlockSpec((blk, H), lambda i: (i, 0)),
        ],
        out_shape=[
            jax.ShapeDtypeStruct((N, HF), jnp.float32),
            jax.ShapeDtypeStruct((N, 2 * H), jnp.float32),
            jax.ShapeDtypeStruct((N, H), jnp.float32),
        ],
    )(x, W, Mcat)


GRP1 = 2
NG1 = E_PAD // (NT * GRP1 * SB)


@functools.partial(
    pl.kernel,
    out_type=[
        jax.ShapeDtypeStruct((H, E_PAD), jnp.float32),
        jax.ShapeDtypeStruct((NC, NPAD, 16), jnp.float32),
    ],
    mesh=_mesh,
    compiler_params=_sc_params,
    scratch_types=[
        pltpu.VMEM_SHARED((NPAD, 16), jnp.float32),
        pltpu.VMEM((4, GRP1, SB), jnp.int32),
        pltpu.VMEM((4, GRP1, SB), jnp.int32),
        pltpu.VMEM((2, GRP1 * SB, 16), jnp.float32),
        pltpu.VMEM((2, GRP1 * SB, 16), jnp.float32),
        pltpu.VMEM((2, GRP1 * SB, 16), jnp.float32),
        pltpu.VMEM((2, H, GRP1 * SB), jnp.float32),
        pltpu.SemaphoreType.DMA, pltpu.SemaphoreType.DMA,
        pltpu.SemaphoreType.DMA, pltpu.SemaphoreType.DMA,
        pltpu.SemaphoreType.DMA, pltpu.SemaphoreType.DMA,
        pltpu.SemaphoreType.DMA, pltpu.SemaphoreType.DMA,
        pltpu.SemaphoreType.DMA, pltpu.SemaphoreType.DMA,
    ],
)
def _sc_pass0(ad_hbm, src_hbm, dst_hbm, zero16_hbm, wt_hbm, s_hbm,
              s_sh, srcb, dstb, arows, brows, wgrp, wtb,
              l0, l1, l2, l3, ga, gb, sa, sb_, wa, wb):
    c = lax.axis_index("c")
    s = lax.axis_index("s")
    tid = c * NS + s
    semL = [l0, l1, l2, l3]
    semG = [ga, gb]
    semS = [sa, sb_]
    semW = [wa, wb]
    lane = lax.iota(jnp.int32, 16)
    EG = GRP1 * SB

    pltpu.sync_copy(zero16_hbm, s_sh.at[pl.ds(s * NPT, NPT)])
    pltpu.sync_copy(zero16_hbm.at[pl.ds(0, EG)], wgrp.at[0])
    pltpu.sync_copy(zero16_hbm.at[pl.ds(0, EG)], wgrp.at[1])
    plsc.subcore_barrier()

    def gsb0_of(g):
        return (tid * NG1 + g) * GRP1

    def issue_lin(g, d4):
        pltpu.async_copy(src_hbm.at[pl.ds(gsb0_of(g), GRP1)], srcb.at[d4],
                         semL[d4])
        pltpu.async_copy(dst_hbm.at[pl.ds(gsb0_of(g), GRP1)], dstb.at[d4],
                         semL[d4])

    def drain_lin(d4):
        pltpu.make_async_copy(src_hbm.at[pl.ds(0, GRP1)], srcb.at[d4],
                              semL[d4]).wait()
        pltpu.make_async_copy(dst_hbm.at[pl.ds(0, GRP1)], dstb.at[d4],
                              semL[d4]).wait()

    def issue_gathers(d4, d2):
        def j_body(j, carry):
            pltpu.async_copy(ad_hbm.at[srcb.at[d4, j]],
                             arows.at[d2, pl.ds(j * SB, SB)], semG[d2])
            pltpu.async_copy(ad_hbm.at[dstb.at[d4, j]],
                             brows.at[d2, pl.ds(j * SB, SB)], semG[d2])
            return carry
        lax.fori_loop(0, GRP1, j_body, 0)

    def drain_gathers(d2):
        for j in range(GRP1):
            pltpu.make_async_copy(ad_hbm.at[srcb.at[0, j]],
                                  arows.at[d2, pl.ds(j * SB, SB)],
                                  semG[d2]).wait()
            pltpu.make_async_copy(ad_hbm.at[dstb.at[0, j]],
                                  brows.at[d2, pl.ds(j * SB, SB)],
                                  semG[d2]).wait()

    def compute(g, d2):
        gsb0 = gsb0_of(g)

        def j_body(j, carry):
            realf = jnp.where(gsb0 + j < SB_REAL, 1.0, 0.0).astype(jnp.float32)
            for ch in range(SB // 16):
                ridx = lane + (j * SB + ch * 16)
                for h in range(H):
                    hcol = jnp.full((16,), h, jnp.int32)
                    av = plsc.load_gather(arows.at[d2], [ridx, hcol])
                    bv = plsc.load_gather(brows.at[d2], [ridx, hcol + H])
                    e = av + bv
                    w = jnp.exp(jnp.maximum(e, 0.2 * e))
                    wtb[d2, h, pl.ds(j * SB + ch * 16, 16)] = w
                    plsc.store_scatter(wgrp.at[d2], [ridx, hcol], w * realf)
            return carry

        lax.fori_loop(0, GRP1, j_body, 0)

    def issue_out(g, d4, d2):
        goff = gsb0_of(g) * SB

        def j_body(j, carry):
            pltpu.async_copy(wgrp.at[d2, pl.ds(j * SB, SB)],
                             s_sh.at[dstb.at[d4, j]], semS[d2], add=True)
            return carry
        lax.fori_loop(0, GRP1, j_body, 0)
        for h in range(H):
            pltpu.async_copy(wtb.at[d2, h], wt_hbm.at[h, pl.ds(goff, EG)],
                             semW[d2])

    def drain_out(d2):
        for j in range(GRP1):
            pltpu.make_async_copy(wgrp.at[d2, pl.ds(j * SB, SB)],
                                  s_sh.at[dstb.at[0, j]], semS[d2]).wait()
        for h in range(H):
            pltpu.make_async_copy(wtb.at[d2, h],
                                  wt_hbm.at[h, pl.ds(0, EG)], semW[d2]).wait()

    issue_lin(0, 0)
    issue_lin(1, 1)
    drain_lin(0)
    issue_gathers(0, 0)

    n_outer = (NG1 + 2 + 3) // 4

    def outer(o, carry):
        for db in range(4):
            g = o * 4 + db
            d2, d4 = db % 2, db
            d2n, d4n = (db + 1) % 2, (db + 1) % 4
            d4nn = (db + 2) % 4

            @pl.when(jnp.logical_and(g >= 2, g < NG1 + 2))
            def _():
                drain_out(d2)

            @pl.when(g + 2 < NG1)
            def _():
                issue_lin(g + 2, d4nn)

            @pl.when(g < NG1)
            def _():
                drain_gathers(d2)

            @pl.when(g + 1 < NG1)
            def _():
                drain_lin(d4n)
                issue_gathers(d4n, d2n)

            @pl.when(g < NG1)
            def _():
                compute(g, d2)
                issue_out(g, d4, d2)
        return carry

    lax.fori_loop(0, n_outer, outer, 0)

    plsc.subcore_barrier()
    pltpu.sync_copy(s_sh.at[pl.ds(s * NPT, NPT)],
                    s_hbm.at[c, pl.ds(s * NPT, NPT)])


@functools.partial(
    pl.kernel,
    out_type=jax.ShapeDtypeStruct((2 * NC, NPAD, 16), jnp.float32),
    mesh=_mesh,
    compiler_params=_sc_params,
    scratch_types=[
        pltpu.VMEM_SHARED((NPAD, 16), jnp.float32),
        pltpu.VMEM((8, GRP2, SB), jnp.int32),
        pltpu.VMEM((8, GRP2, SB), jnp.int32),
        pltpu.VMEM((4, GRP2 * SB), jnp.float32),
        pltpu.VMEM((4, GRP2 * SB), jnp.float32),
        pltpu.VMEM((4, GRP2 * SB, 16), jnp.float32),
        pltpu.VMEM((2, GRP2 * SB, 16), jnp.float32),
        pltpu.SemaphoreType.DMA((4,)),
        pltpu.SemaphoreType.DMA((4,)),
        pltpu.SemaphoreType.DMA((2,)),
    ],
)
def _sc_main(hp_hbm, src_hbm, dst_hbm, wt_hbm, zero16_hbm, acc_hbm,
             acc_sh, srcb, dstb, w0b, w1b, hrows, msgb,
             semLs, semGs, semSs):
    c = lax.axis_index("c")
    s = lax.axis_index("s")
    lane = lax.iota(jnp.int32, 16)
    EG = GRP2 * SB

    def gsb0_of(g):
        return (s * NG2 + g) * GRP2

    for p_local in range(2):
        pair = c * 2 + p_local
        h0 = 2 * pair
        h1 = h0 + 1

        pltpu.sync_copy(zero16_hbm, acc_sh.at[pl.ds(s * NPT, NPT)])
        plsc.subcore_barrier()

        def issue_lin(g, s8, s4):
            gsb0 = gsb0_of(g)
            goff = gsb0 * SB
            pltpu.async_copy(src_hbm.at[pl.ds(gsb0, GRP2)], srcb.at[s8],
                             semLs.at[s4])
            pltpu.async_copy(dst_hbm.at[pl.ds(gsb0, GRP2)], dstb.at[s8],
                             semLs.at[s4])
            pltpu.async_copy(wt_hbm.at[h0, pl.ds(goff, EG)], w0b.at[s4],
                             semLs.at[s4])
            pltpu.async_copy(wt_hbm.at[h1, pl.ds(goff, EG)], w1b.at[s4],
                             semLs.at[s4])

        def drain_lin(s8, s4):
            pltpu.make_async_copy(src_hbm.at[pl.ds(0, GRP2)], srcb.at[s8],
                                  semLs.at[s4]).wait()
            pltpu.make_async_copy(dst_hbm.at[pl.ds(0, GRP2)], dstb.at[s8],
                                  semLs.at[s4]).wait()
            for _ in range(2):
                pltpu.make_async_copy(wt_hbm.at[0, pl.ds(0, EG)], w0b.at[s4],
                                      semLs.at[s4]).wait()

        def compute_idx(s8):
            def j_body(j, carry):
                for ch in range(SB // 16):
                    sv = srcb[s8, j, pl.ds(ch * 16, 16)]
                    srcb[s8, j, pl.ds(ch * 16, 16)] = sv * 4 + pair
                return carry
            lax.fori_loop(0, GRP2, j_body, 0)

        def issue_gathers(s8, h4):
            def j_body(j, carry):
                pltpu.async_copy(hp_hbm.at[srcb.at[s8, j]],
                                 hrows.at[h4, pl.ds(j * SB, SB)], semGs.at[h4])
                return carry
            lax.fori_loop(0, GRP2, j_body, 0)

        def drain_gathers(h4):
            for j in range(GRP2):
                pltpu.make_async_copy(hp_hbm.at[srcb.at[0, j]],
                                      hrows.at[h4, pl.ds(j * SB, SB)],
                                      semGs.at[h4]).wait()

        def compute(g, s4, h4, m2):
            gsb0 = gsb0_of(g)

            def j_body(j, carry):
                realf = jnp.where(gsb0 + j < SB_REAL, 1.0,
                                  0.0).astype(jnp.float32)
                for ch in range(SB // 16):
                    ridx = lane + (j * SB + ch * 16)
                    w0v = w0b[s4, pl.ds(j * SB + ch * 16, 16)] * realf
                    w1v = w1b[s4, pl.ds(j * SB + ch * 16, 16)] * realf
                    for f in range(16):
                        fcol = jnp.full((16,), f, jnp.int32)
                        hv = plsc.load_gather(hrows.at[h4], [ridx, fcol])
                        m = hv * (w0v if f < 8 else w1v)
                        plsc.store_scatter(msgb.at[m2], [ridx, fcol], m)
                return carry

            lax.fori_loop(0, GRP2, j_body, 0)

        def issue_scatters(s8, m2):
            def j_body(j, carry):
                pltpu.async_copy(msgb.at[m2, pl.ds(j * SB, SB)],
                                 acc_sh.at[dstb.at[s8, j]], semSs.at[m2], add=True)
                return carry
            lax.fori_loop(0, GRP2, j_body, 0)

        def drain_scatters(m2):
            for j in range(GRP2):
                pltpu.make_async_copy(msgb.at[m2, pl.ds(j * SB, SB)],
                                      acc_sh.at[dstb.at[0, j]],
                                      semSs.at[m2]).wait()

        issue_lin(0, 0, 0)
        issue_lin(1, 1, 1)
        issue_lin(2, 2, 2)
        drain_lin(0, 0)
        compute_idx(0)
        issue_gathers(0, 0)
        drain_lin(1, 1)
        compute_idx(1)
        issue_gathers(1, 1)

        def body(g, carry):
            m2 = lax.rem(g, 2)
            s4 = lax.rem(g, 4)
            s8 = lax.rem(g, 8)
            h4 = s4
            s8_3, s4_3 = lax.rem(g + 3, 8), lax.rem(g + 3, 4)
            s8_2, s4_2 = lax.rem(g + 2, 8), lax.rem(g + 2, 4)

            @pl.when(g >= 2)
            def _():
                drain_scatters(m2)

            @pl.when(g + 3 < NG2)
            def _():
                issue_lin(g + 3, s8_3, s4_3)

            @pl.when(g + 2 < NG2)
            def _():
                drain_lin(s8_2, s4_2)
                compute_idx(s8_2)
                issue_gathers(s8_2, s4_2)

            @pl.when(g < NG2)
            def _():
                drain_gathers(h4)
                compute(g, s4, h4, m2)
                issue_scatters(s8, m2)
            return carry

        lax.fori_loop(0, NG2 + 2, body, 0)

        plsc.subcore_barrier()
        pltpu.sync_copy(acc_sh.at[pl.ds(s * NPT, NPT)],
                        acc_hbm.at[pair, pl.ds(s * NPT, NPT)])
        plsc.subcore_barrier()


def _combine_body(a0_ref, a1_ref, a2_ref, a3_ref, s0_ref, s1_ref, wl_ref,
                  h_ref, r_ref, b_ref, out_ref):
    s_tot = s0_ref[:, :H] + s1_ref[:, :H] + wl_ref[...]
    r = r_ref[...]
    s64 = jnp.dot(s_tot, r, preferred_element_type=jnp.float32)
    wl64 = jnp.dot(wl_ref[...], r, preferred_element_type=jnp.float32)
    acc = jnp.concatenate(
        [a0_ref[...], a1_ref[...], a2_ref[...], a3_ref[...]], axis=1)
    out_ref[...] = (acc + wl64 * h_ref[...]) / (s64 + 1e-16) + b_ref[...]


def _combine(a0, a1, a2, a3, s0, s1, wl, h, R, bias2d):
    blk = 1000
    node16 = pl.BlockSpec((blk, 16), lambda i: (i, 0))
    node8 = pl.BlockSpec((blk, H), lambda i: (i, 0))
    return pl.pallas_call(
        _combine_body,
        grid=(N // blk,),
        in_specs=[
            node16, node16, node16, node16,
            node16, node16, node8,
            pl.BlockSpec((blk, HF), lambda i: (i, 0)),
            pl.BlockSpec((H, HF), lambda i: (0, 0)),
            pl.BlockSpec((1, HF), lambda i: (0, 0)),
        ],
        out_specs=pl.BlockSpec((blk, HF), lambda i: (i, 0)),
        out_shape=jax.ShapeDtypeStruct((N, HF), jnp.float32),
    )(a0, a1, a2, a3, s0, s1, wl, h, R, bias2d)


def kernel(x, edge_index, W, att_src, att_dst, bias):
    f32 = jnp.float32
    a_s = att_src.reshape(H, F).astype(f32)
    a_d = att_dst.reshape(H, F).astype(f32)
    eye = jnp.eye(H, dtype=f32)
    Ms = (a_s[:, :, None] * eye[:, None, :]).reshape(HF, H)
    Md = (a_d[:, :, None] * eye[:, None, :]).reshape(HF, H)
    Mcat = jnp.concatenate([Ms, Md], axis=1)
    R = jnp.repeat(eye, F, axis=1)

    pad = E_PAD - E
    src_p = jnp.concatenate(
        [edge_index[0], jnp.zeros((pad,), jnp.int32)]).reshape(-1, SB)
    dst_p = jnp.concatenate(
        [edge_index[1], jnp.zeros((pad,), jnp.int32)]).reshape(-1, SB)

    h, ad, wl = _dense_prologue(x.astype(f32), W.astype(f32), Mcat)
    zeros16 = jnp.zeros((NPT, 16), f32)

    wt, s_part = _sc_pass0(ad, src_p, dst_p, zeros16)
    hp = h.reshape(N * 4, 16)
    acc = _sc_main(hp, src_p, dst_p, wt, zeros16)

    out = _combine(acc[0], acc[1], acc[2], acc[3],
                   s_part[0], s_part[1], wl, h, R,
                   bias.astype(f32).reshape(1, HF))
    return out


# speedup vs baseline: 83.3292x; 1.0120x over previous
import functools

import jax
import jax.numpy as jnp
from jax import lax
from jax.experimental import pallas as pl
from jax.experimental.pallas import tpu as pltpu
from jax.experimental.pallas import tpu_sc as plsc

N = 100000
E = 1600000
D_IN = 34
H = 8
F = 8
HF = H * F

NC, NS = 2, 16
NT = NC * NS
SB = 128
E_PAD = NT * 49 * 1024
SB_REAL = E // SB
NSB = E_PAD // SB
NPT = 6256
NPAD = NS * NPT

GRP1 = 4
NG1 = E_PAD // (NT * GRP1 * SB)
GRP2 = 2
NG2 = E_PAD // (NS * GRP2 * SB)

_mesh = plsc.VectorSubcoreMesh(core_axis_name="c", subcore_axis_name="s",
                               num_cores=NC, num_subcores=NS)
_sc_params = pltpu.CompilerParams(needs_layout_passes=False,
                                  use_tc_tiling_on_sc=False)


def _dense_body(x_ref, w_ref, m_ref, h_ref, ad_ref, wl_ref):
    h = jnp.dot(x_ref[...], w_ref[...], preferred_element_type=jnp.float32)
    h_ref[...] = h
    ad = jnp.dot(h, m_ref[...], preferred_element_type=jnp.float32)
    ad_ref[...] = ad
    e = ad[:, :H] + ad[:, H:]
    wl_ref[...] = jnp.exp(jnp.maximum(e, 0.2 * e))


def _dense_prologue(x, W, Mcat):
    blk = 1000
    return pl.pallas_call(
        _dense_body,
        grid=(N // blk,),
        in_specs=[
            pl.BlockSpec((blk, D_IN), lambda i: (i, 0)),
            pl.BlockSpec((D_IN, HF), lambda i: (0, 0)),
            pl.BlockSpec((HF, 2 * H), lambda i: (0, 0)),
        ],
        out_specs=[
            pl.BlockSpec((blk, HF), lambda i: (i, 0)),
            pl.BlockSpec((blk, 2 * H), lambda i: (i, 0)),
            pl.BlockSpec((blk, H), lambda i: (i, 0)),
        ],
        out_shape=[
            jax.ShapeDtypeStruct((N, HF), jnp.float32),
            jax.ShapeDtypeStruct((N, 2 * H), jnp.float32),
            jax.ShapeDtypeStruct((N, H), jnp.float32),
        ],
    )(x, W, Mcat)


GRP1 = 2
NG1 = E_PAD // (NT * GRP1 * SB)


@functools.partial(
    pl.kernel,
    out_type=[
        jax.ShapeDtypeStruct((H, E_PAD), jnp.float32),
        jax.ShapeDtypeStruct((NC, NPAD, H), jnp.float32),
    ],
    mesh=_mesh,
    compiler_params=_sc_params,
    scratch_types=[
        pltpu.VMEM_SHARED((NPAD, H), jnp.float32),
        pltpu.VMEM((4, GRP1, SB), jnp.int32),
        pltpu.VMEM((4, GRP1, SB), jnp.int32),
        pltpu.VMEM((2, GRP1 * SB, 16), jnp.float32),
        pltpu.VMEM((2, GRP1 * SB, 16), jnp.float32),
        pltpu.VMEM((2, GRP1 * SB, H), jnp.float32),
        pltpu.VMEM((2, H, GRP1 * SB), jnp.float32),
        pltpu.SemaphoreType.DMA, pltpu.SemaphoreType.DMA,
        pltpu.SemaphoreType.DMA, pltpu.SemaphoreType.DMA,
        pltpu.SemaphoreType.DMA, pltpu.SemaphoreType.DMA,
        pltpu.SemaphoreType.DMA, pltpu.SemaphoreType.DMA,
        pltpu.SemaphoreType.DMA, pltpu.SemaphoreType.DMA,
    ],
)
def _sc_pass0(ad_hbm, src_hbm, dst_hbm, zero8_hbm, wt_hbm, s_hbm,
              s_sh, srcb, dstb, arows, brows, wgrp, wtb,
              l0, l1, l2, l3, ga, gb, sa, sb_, wa, wb):
    c = lax.axis_index("c")
    s = lax.axis_index("s")
    tid = c * NS + s
    semL = [l0, l1, l2, l3]
    semG = [ga, gb]
    semS = [sa, sb_]
    semW = [wa, wb]
    lane = lax.iota(jnp.int32, 16)
    EG = GRP1 * SB

    pltpu.sync_copy(zero8_hbm, s_sh.at[pl.ds(s * NPT, NPT)])
    plsc.subcore_barrier()

    def gsb0_of(g):
        return (tid * NG1 + g) * GRP1

    def issue_lin(g, d4):
        pltpu.async_copy(src_hbm.at[pl.ds(gsb0_of(g), GRP1)], srcb.at[d4],
                         semL[d4])
        pltpu.async_copy(dst_hbm.at[pl.ds(gsb0_of(g), GRP1)], dstb.at[d4],
                         semL[d4])

    def drain_lin(d4):
        pltpu.make_async_copy(src_hbm.at[pl.ds(0, GRP1)], srcb.at[d4],
                              semL[d4]).wait()
        pltpu.make_async_copy(dst_hbm.at[pl.ds(0, GRP1)], dstb.at[d4],
                              semL[d4]).wait()

    def issue_gathers(d4, d2):
        def j_body(j, carry):
            pltpu.async_copy(ad_hbm.at[srcb.at[d4, j]],
                             arows.at[d2, pl.ds(j * SB, SB)], semG[d2])
            pltpu.async_copy(ad_hbm.at[dstb.at[d4, j]],
                             brows.at[d2, pl.ds(j * SB, SB)], semG[d2])
            return carry
        lax.fori_loop(0, GRP1, j_body, 0)

    def drain_gathers(d2):
        for j in range(GRP1):
            pltpu.make_async_copy(ad_hbm.at[srcb.at[0, j]],
                                  arows.at[d2, pl.ds(j * SB, SB)],
                                  semG[d2]).wait()
            pltpu.make_async_copy(ad_hbm.at[dstb.at[0, j]],
                                  brows.at[d2, pl.ds(j * SB, SB)],
                                  semG[d2]).wait()

    def compute(g, d2):
        gsb0 = gsb0_of(g)

        def j_body(j, carry):
            realf = jnp.where(gsb0 + j < SB_REAL, 1.0, 0.0).astype(jnp.float32)
            for ch in range(SB // 16):
                ridx = lane + (j * SB + ch * 16)
                for h in range(H):
                    hcol = jnp.full((16,), h, jnp.int32)
                    av = plsc.load_gather(arows.at[d2], [ridx, hcol])
                    bv = plsc.load_gather(brows.at[d2], [ridx, hcol + H])
                    e = av + bv
                    w = jnp.exp(jnp.maximum(e, 0.2 * e))
                    wtb[d2, h, pl.ds(j * SB + ch * 16, 16)] = w
                    plsc.store_scatter(wgrp.at[d2], [ridx, hcol], w * realf)
            return carry

        lax.fori_loop(0, GRP1, j_body, 0)

    def issue_out(g, d4, d2):
        goff = gsb0_of(g) * SB

        def j_body(j, carry):
            pltpu.async_copy(wgrp.at[d2, pl.ds(j * SB, SB)],
                             s_sh.at[dstb.at[d4, j]], semS[d2], add=True)
            return carry
        lax.fori_loop(0, GRP1, j_body, 0)
        for h in range(H):
            pltpu.async_copy(wtb.at[d2, h], wt_hbm.at[h, pl.ds(goff, EG)],
                             semW[d2])

    def drain_out(d2):
        for j in range(GRP1):
            pltpu.make_async_copy(wgrp.at[d2, pl.ds(j * SB, SB)],
                                  s_sh.at[dstb.at[0, j]], semS[d2]).wait()
        for h in range(H):
            pltpu.make_async_copy(wtb.at[d2, h],
                                  wt_hbm.at[h, pl.ds(0, EG)], semW[d2]).wait()

    issue_lin(0, 0)
    issue_lin(1, 1)
    drain_lin(0)
    issue_gathers(0, 0)

    n_outer = (NG1 + 2 + 3) // 4

    def outer(o, carry):
        for db in range(4):
            g = o * 4 + db
            d2, d4 = db % 2, db
            d2n, d4n = (db + 1) % 2, (db + 1) % 4
            d4nn = (db + 2) % 4

            @pl.when(jnp.logical_and(g >= 2, g < NG1 + 2))
            def _():
                drain_out(d2)

            @pl.when(g + 2 < NG1)
            def _():
                issue_lin(g + 2, d4nn)

            @pl.when(g < NG1)
            def _():
                drain_gathers(d2)

            @pl.when(g + 1 < NG1)
            def _():
                drain_lin(d4n)
                issue_gathers(d4n, d2n)

            @pl.when(g < NG1)
            def _():
                compute(g, d2)
                issue_out(g, d4, d2)
        return carry

    lax.fori_loop(0, n_outer, outer, 0)

    plsc.subcore_barrier()
    pltpu.sync_copy(s_sh.at[pl.ds(s * NPT, NPT)],
                    s_hbm.at[c, pl.ds(s * NPT, NPT)])


@functools.partial(
    pl.kernel,
    out_type=jax.ShapeDtypeStruct((2 * NC, NPAD, 16), jnp.float32),
    mesh=_mesh,
    compiler_params=_sc_params,
    scratch_types=[
        pltpu.VMEM_SHARED((NPAD, 16), jnp.float32),
        pltpu.VMEM((8, GRP2, SB), jnp.int32),
        pltpu.VMEM((8, GRP2, SB), jnp.int32),
        pltpu.VMEM((4, GRP2 * SB), jnp.float32),
        pltpu.VMEM((4, GRP2 * SB), jnp.float32),
        pltpu.VMEM((4, GRP2 * SB, 16), jnp.float32),
        pltpu.VMEM((2, GRP2 * SB, 16), jnp.float32),
        pltpu.SemaphoreType.DMA((4,)),
        pltpu.SemaphoreType.DMA((4,)),
        pltpu.SemaphoreType.DMA((2,)),
    ],
)
def _sc_main(hp_hbm, src_hbm, dst_hbm, wt_hbm, zero16_hbm, acc_hbm,
             acc_sh, srcb, dstb, w0b, w1b, hrows, msgb,
             semLs, semGs, semSs):
    c = lax.axis_index("c")
    s = lax.axis_index("s")
    lane = lax.iota(jnp.int32, 16)
    EG = GRP2 * SB

    def gsb0_of(g):
        return (s * NG2 + g) * GRP2

    for p_local in range(2):
        pair = c * 2 + p_local
        h0 = 2 * pair
        h1 = h0 + 1

        pltpu.sync_copy(zero16_hbm, acc_sh.at[pl.ds(s * NPT, NPT)])
        plsc.subcore_barrier()

        def issue_lin(g, s8, s4):
            gsb0 = gsb0_of(g)
            goff = gsb0 * SB
            pltpu.async_copy(src_hbm.at[pl.ds(gsb0, GRP2)], srcb.at[s8],
                             semLs.at[s4])
            pltpu.async_copy(dst_hbm.at[pl.ds(gsb0, GRP2)], dstb.at[s8],
                             semLs.at[s4])
            pltpu.async_copy(wt_hbm.at[h0, pl.ds(goff, EG)], w0b.at[s4],
                             semLs.at[s4])
            pltpu.async_copy(wt_hbm.at[h1, pl.ds(goff, EG)], w1b.at[s4],
                             semLs.at[s4])

        def drain_lin(s8, s4):
            pltpu.make_async_copy(src_hbm.at[pl.ds(0, GRP2)], srcb.at[s8],
                                  semLs.at[s4]).wait()
            pltpu.make_async_copy(dst_hbm.at[pl.ds(0, GRP2)], dstb.at[s8],
                                  semLs.at[s4]).wait()
            for _ in range(2):
                pltpu.make_async_copy(wt_hbm.at[0, pl.ds(0, EG)], w0b.at[s4],
                                      semLs.at[s4]).wait()

        def compute_idx(s8):
            def j_body(j, carry):
                for ch in range(SB // 16):
                    sv = srcb[s8, j, pl.ds(ch * 16, 16)]
                    srcb[s8, j, pl.ds(ch * 16, 16)] = sv * 4 + pair
                return carry
            lax.fori_loop(0, GRP2, j_body, 0)

        def issue_gathers(s8, h4):
            def j_body(j, carry):
                pltpu.async_copy(hp_hbm.at[srcb.at[s8, j]],
                                 hrows.at[h4, pl.ds(j * SB, SB)], semGs.at[h4])
                return carry
            lax.fori_loop(0, GRP2, j_body, 0)

        def drain_gathers(h4):
            for j in range(GRP2):
                pltpu.make_async_copy(hp_hbm.at[srcb.at[0, j]],
                                      hrows.at[h4, pl.ds(j * SB, SB)],
                                      semGs.at[h4]).wait()

        def compute(g, s4, h4, m2):
            gsb0 = gsb0_of(g)

            def j_body(j, carry):
                realf = jnp.where(gsb0 + j < SB_REAL, 1.0,
                                  0.0).astype(jnp.float32)
                for ch in range(SB // 16):
                    ridx = lane + (j * SB + ch * 16)
                    w0v = w0b[s4, pl.ds(j * SB + ch * 16, 16)] * realf
                    w1v = w1b[s4, pl.ds(j * SB + ch * 16, 16)] * realf
                    for f in range(16):
                        fcol = jnp.full((16,), f, jnp.int32)
                        hv = plsc.load_gather(hrows.at[h4], [ridx, fcol])
                        m = hv * (w0v if f < 8 else w1v)
                        plsc.store_scatter(msgb.at[m2], [ridx, fcol], m)
                return carry

            lax.fori_loop(0, GRP2, j_body, 0)

        def issue_scatters(s8, m2):
            def j_body(j, carry):
                pltpu.async_copy(msgb.at[m2, pl.ds(j * SB, SB)],
                                 acc_sh.at[dstb.at[s8, j]], semSs.at[m2], add=True)
                return carry
            lax.fori_loop(0, GRP2, j_body, 0)

        def drain_scatters(m2):
            for j in range(GRP2):
                pltpu.make_async_copy(msgb.at[m2, pl.ds(j * SB, SB)],
                                      acc_sh.at[dstb.at[0, j]],
                                      semSs.at[m2]).wait()

        issue_lin(0, 0, 0)
        issue_lin(1, 1, 1)
        issue_lin(2, 2, 2)
        drain_lin(0, 0)
        compute_idx(0)
        issue_gathers(0, 0)
        drain_lin(1, 1)
        compute_idx(1)
        issue_gathers(1, 1)

        def body(g, carry):
            m2 = lax.rem(g, 2)
            s4 = lax.rem(g, 4)
            s8 = lax.rem(g, 8)
            h4 = s4
            s8_3, s4_3 = lax.rem(g + 3, 8), lax.rem(g + 3, 4)
            s8_2, s4_2 = lax.rem(g + 2, 8), lax.rem(g + 2, 4)

            @pl.when(g >= 2)
            def _():
                drain_scatters(m2)

            @pl.when(g + 3 < NG2)
            def _():
                issue_lin(g + 3, s8_3, s4_3)

            @pl.when(g + 2 < NG2)
            def _():
                drain_lin(s8_2, s4_2)
                compute_idx(s8_2)
                issue_gathers(s8_2, s4_2)

            @pl.when(g < NG2)
            def _():
                drain_gathers(h4)
                compute(g, s4, h4, m2)
                issue_scatters(s8, m2)
            return carry

        lax.fori_loop(0, NG2 + 2, body, 0)

        plsc.subcore_barrier()
        pltpu.sync_copy(acc_sh.at[pl.ds(s * NPT, NPT)],
                        acc_hbm.at[pair, pl.ds(s * NPT, NPT)])
        plsc.subcore_barrier()


def _combine_body(a0_ref, a1_ref, a2_ref, a3_ref, s0_ref, s1_ref, wl_ref,
                  h_ref, r_ref, b_ref, out_ref):
    s_tot = s0_ref[...] + s1_ref[...] + wl_ref[...]
    r = r_ref[...]
    s64 = jnp.dot(s_tot, r, preferred_element_type=jnp.float32)
    wl64 = jnp.dot(wl_ref[...], r, preferred_element_type=jnp.float32)
    acc = jnp.concatenate(
        [a0_ref[...], a1_ref[...], a2_ref[...], a3_ref[...]], axis=1)
    out_ref[...] = (acc + wl64 * h_ref[...]) / (s64 + 1e-16) + b_ref[...]


def _combine(a0, a1, a2, a3, s0, s1, wl, h, R, bias2d):
    blk = 1000
    node16 = pl.BlockSpec((blk, 16), lambda i: (i, 0))
    node8 = pl.BlockSpec((blk, H), lambda i: (i, 0))
    return pl.pallas_call(
        _combine_body,
        grid=(N // blk,),
        in_specs=[
            node16, node16, node16, node16,
            node8, node8, node8,
            pl.BlockSpec((blk, HF), lambda i: (i, 0)),
            pl.BlockSpec((H, HF), lambda i: (0, 0)),
            pl.BlockSpec((1, HF), lambda i: (0, 0)),
        ],
        out_specs=pl.BlockSpec((blk, HF), lambda i: (i, 0)),
        out_shape=jax.ShapeDtypeStruct((N, HF), jnp.float32),
    )(a0, a1, a2, a3, s0, s1, wl, h, R, bias2d)


def kernel(x, edge_index, W, att_src, att_dst, bias):
    f32 = jnp.float32
    a_s = att_src.reshape(H, F).astype(f32)
    a_d = att_dst.reshape(H, F).astype(f32)
    eye = jnp.eye(H, dtype=f32)
    Ms = (a_s[:, :, None] * eye[:, None, :]).reshape(HF, H)
    Md = (a_d[:, :, None] * eye[:, None, :]).reshape(HF, H)
    Mcat = jnp.concatenate([Ms, Md], axis=1)
    R = jnp.repeat(eye, F, axis=1)

    pad = E_PAD - E
    src_p = jnp.concatenate(
        [edge_index[0], jnp.zeros((pad,), jnp.int32)]).reshape(-1, SB)
    dst_p = jnp.concatenate(
        [edge_index[1], jnp.zeros((pad,), jnp.int32)]).reshape(-1, SB)

    h, ad, wl = _dense_prologue(x.astype(f32), W.astype(f32), Mcat)
    zeros8 = jnp.zeros((NPT, H), f32)
    zeros16 = jnp.zeros((NPT, 16), f32)

    wt, s_part = _sc_pass0(ad, src_p, dst_p, zeros8)
    hp = h.reshape(N * 4, 16)
    acc = _sc_main(hp, src_p, dst_p, wt, zeros16)

    out = _combine(acc[0], acc[1], acc[2], acc[3],
                   s_part[0], s_part[1], wl, h, R,
                   bias.astype(f32).reshape(1, HF))
    return out
